# baseline (speedup 1.0000x reference)
import functools

import jax
import jax.numpy as jnp
from jax import lax
from jax.experimental import pallas as pl
from jax.experimental.pallas import tpu as pltpu
from jax.experimental.pallas import tpu_sc as plsc

N_NODES = 10000
DIM = 128
HALF = DIM // 2
NC, NS = 2, 16
CHUNK = 64
DW = 80
N_PAD = 10112
RPT = N_PAD // NS
RPT_LAST = N_NODES - (NS - 1) * RPT
PASS_CHUNKS = 40


def _transform_body(x_ref, w_ref, b_ref, o_ref):
    blk = x_ref.shape[0]
    prop = lax.dot_general(
        x_ref[...], w_ref[...],
        dimension_numbers=(((1,), (1,)), ((), ())),
        preferred_element_type=jnp.float32,
    ) + b_ref[...]
    col = lax.broadcasted_iota(jnp.int32, (blk, DW - HALF), 1)
    cnt_pad = jnp.where(col == 0, jnp.float32(1.0), jnp.float32(0.0))
    o_ref[0] = jnp.concatenate([prop[:, :HALF], cnt_pad], axis=1)
    o_ref[1] = jnp.concatenate([prop[:, HALF:], cnt_pad], axis=1)


def _transform(node_states, w, b):
    blk = 1264
    grid = N_PAD // blk
    return pl.pallas_call(
        _transform_body,
        grid=(grid,),
        in_specs=[
            pl.BlockSpec((blk, DIM), lambda i: (i, 0)),
            pl.BlockSpec((DIM, DIM), lambda i: (0, 0)),
            pl.BlockSpec((1, DIM), lambda i: (0, 0)),
        ],
        out_specs=pl.BlockSpec((NC, blk, DW), lambda i: (0, i, 0)),
        out_shape=jax.ShapeDtypeStruct((NC, N_PAD, DW), jnp.float32),
    )(node_states, w, b.reshape(1, DIM))


def _make_scatter(cpt):
    mesh = plsc.VectorSubcoreMesh(core_axis_name="c", subcore_axis_name="s",
                                  num_cores=NC, num_subcores=NS)
    npass = cpt // PASS_CHUNKS

    @functools.partial(
        pl.kernel,
        out_type=jax.ShapeDtypeStruct((N_NODES, DIM), jnp.float32),
        mesh=mesh,
        scratch_types=[
            pltpu.VMEM((PASS_CHUNKS, 2, CHUNK), jnp.int32),
            pltpu.VMEM((CHUNK, DW), jnp.float32),
            pltpu.VMEM((CHUNK, DW), jnp.float32),
            pltpu.VMEM((CHUNK, HALF), jnp.float32),
            pltpu.VMEM_SHARED((N_PAD, DW), jnp.float32),
            pltpu.VMEM_SHARED((N_PAD, DW), jnp.float32),
            pltpu.SemaphoreType.DMA,
            pltpu.SemaphoreType.DMA,
        ],
        compiler_params=pltpu.CompilerParams(use_tc_tiling_on_sc=False),
    )
    def scatter_kernel(st_hbm, prop_hbm, out_hbm,
                       idx_blk, rows_a, rows_b, obuf, tab, acc,
                       sem_a, sem_b):
        c = lax.axis_index("c")
        s = lax.axis_index("s")
        base = s * RPT

        tab_cp = pltpu.async_copy(prop_hbm.at[c, pl.ds(base, RPT)],
                                  tab.at[pl.ds(base, RPT)], sem_b)

        def zrow(r, carry):
            def zcol(cc, carry2):
                rows_a[r, pl.ds(cc * 16, 16)] = jnp.zeros((16,), jnp.float32)
                return carry2
            return lax.fori_loop(0, DW // 16, zcol, carry)
        lax.fori_loop(0, CHUNK, zrow, 0)

        nfull = RPT // CHUNK
        for k in range(nfull):
            pltpu.sync_copy(rows_a, acc.at[pl.ds(base + k * CHUNK, CHUNK)])
        rem = RPT - nfull * CHUNK
        if rem:
            pltpu.sync_copy(rows_a.at[pl.ds(0, rem)],
                            acc.at[pl.ds(base + nfull * CHUNK, rem)])
        tab_cp.wait()
        plsc.subcore_barrier()

        def pass_body(p, carry):
            pltpu.sync_copy(
                st_hbm.at[pl.ds(s * cpt + p * PASS_CHUNKS, PASS_CHUNKS)],
                idx_blk)
            pltpu.async_copy(tab.at[idx_blk.at[0, 0]], rows_a, sem_a)

            def body(kk, c2):
                k = kk * 2
                pltpu.async_copy(tab.at[idx_blk.at[k + 1, 0]], rows_b, sem_b)
                pltpu.make_async_copy(tab.at[idx_blk.at[k, 0]], rows_a,
                                      sem_a).wait()
                pltpu.sync_copy(rows_a, acc.at[idx_blk.at[k, 1]], add=True)

                @pl.when(k + 2 < PASS_CHUNKS)
                def _():
                    pltpu.async_copy(tab.at[idx_blk.at[k + 2, 0]], rows_a,
                                     sem_a)
                pltpu.make_async_copy(tab.at[idx_blk.at[k + 1, 0]], rows_b,
                                      sem_b).wait()
                pltpu.sync_copy(rows_b, acc.at[idx_blk.at[k + 1, 1]],
                                add=True)
                return c2
            lax.fori_loop(0, PASS_CHUNKS // 2, body, 0)
            return carry
        lax.fori_loop(0, npass, pass_body, 0)

        plsc.subcore_barrier()

        one16 = jnp.full((16,), 1.0, jnp.float32)

        def emit_block(r0, nrows):
            pltpu.sync_copy(acc.at[pl.ds(r0, nrows)],
                            rows_a.at[pl.ds(0, nrows)])

            def drow(r, carry):
                cnt = jnp.full(
                    (16,), rows_a[r, pl.ds(HALF, 16)][0], jnp.float32)
                inv = one16 / jnp.maximum(cnt, one16)
                for q in range(HALF // 16):
                    obuf[r, pl.ds(q * 16, 16)] = (
                        rows_a[r, pl.ds(q * 16, 16)] * inv)
                return carry
            lax.fori_loop(0, nrows, drow, 0)
            pltpu.sync_copy(obuf.at[pl.ds(0, nrows)],
                            out_hbm.at[pl.ds(r0, nrows),
                                       pl.ds(c * HALF, HALF)])

        def emit_rows(total):
            nb = total // CHUNK
            for k in range(nb):
                emit_block(base + k * CHUNK, CHUNK)
            tail = total - nb * CHUNK
            if tail:
                emit_block(base + nb * CHUNK, tail)

        @pl.when(s < NS - 1)
        def _():
            emit_rows(RPT)

        @pl.when(s == NS - 1)
        def _():
            emit_rows(RPT_LAST)

    return scatter_kernel


def kernel(edge_lists, node_states, W, b):
    e = edge_lists.shape[1]
    cpt = -(-e // (NS * CHUNK))
    cpt = -(-cpt // PASS_CHUNKS) * PASS_CHUNKS
    e_pad = NS * cpt * CHUNK
    src = edge_lists[0, :, 0].astype(jnp.int32)
    tgt = edge_lists[0, :, 1].astype(jnp.int32)
    pad = e_pad - e
    src = jnp.concatenate([src, jnp.zeros((pad,), jnp.int32)])
    tgt = jnp.concatenate([tgt, jnp.full((pad,), N_NODES, jnp.int32)])
    st = jnp.stack([src.reshape(e_pad // CHUNK, CHUNK),
                    tgt.reshape(e_pad // CHUNK, CHUNK)], axis=1)

    prop = _transform(node_states, W, b)
    return _make_scatter(cpt)(st, prop)
